# Initial kernel scaffold; baseline (speedup 1.0000x reference)
import jax
import jax.numpy as jnp
from jax.experimental import pallas as pl


def kernel(users, items, user_emb, item_emb, user_bias, item_bias, bias):
    raise NotImplementedError("write your pallas kernel here")


# speedup vs baseline: 1.1571x; 1.1571x over previous
import functools

import jax
import jax.numpy as jnp
from jax import lax
from jax.experimental import pallas as pl
from jax.experimental.pallas import tpu as pltpu
from jax.experimental.pallas import tpu_sc as plsc

NC = 2
NS = 16
LANES = 16
IDX_CHUNK = 128


def _build_kernel(B, L, D, S):
    NW = NC * NS
    UPW = B // NW
    PPW = UPW * L
    NSC = PPW // S
    KI = S // IDX_CHUNK
    KU = UPW // IDX_CHUNK
    NG = S // LANES

    mesh = plsc.VectorSubcoreMesh(core_axis_name="c", subcore_axis_name="s")

    @functools.partial(
        pl.kernel,
        mesh=mesh,
        compiler_params=pltpu.CompilerParams(
            needs_layout_passes=False, use_tc_tiling_on_sc=False),
        out_type=jax.ShapeDtypeStruct((B * L,), jnp.float32),
        scratch_types=[
            pltpu.VMEM((UPW,), jnp.int32),
            pltpu.VMEM((UPW, D), jnp.float32),
            pltpu.VMEM((UPW,), jnp.float32),
            pltpu.VMEM((S,), jnp.int32),
            pltpu.VMEM((S, D), jnp.float32),
            pltpu.VMEM((S,), jnp.float32),
            pltpu.VMEM((S,), jnp.float32),
            pltpu.SemaphoreType.DMA,
        ],
    )
    def mf_kernel(users_h, items_h, uemb_h, iemb_h, ubias_h, ibias_h, out_h,
                  uidx_v, uemb_v, ubias_v, idx_v, irows_v, ibias_v, out_v,
                  sem):
        w = lax.axis_index("s") * NC + lax.axis_index("c")
        ubase = w * UPW
        pbase0 = w * PPW

        pltpu.sync_copy(users_h.at[pl.ds(ubase, UPW)], uidx_v)
        handles = []
        for k in range(KU):
            sl = pl.ds(k * IDX_CHUNK, IDX_CHUNK)
            idx = uidx_v.at[sl]
            handles.append(pltpu.async_copy(uemb_h.at[idx], uemb_v.at[sl], sem))
            handles.append(pltpu.async_copy(ubias_h.at[idx], ubias_v.at[sl], sem))
        for h in handles:
            h.wait()

        lid = lax.iota(jnp.int32, 16)

        def superchunk(sc, carry):
            pbase = pbase0 + sc * S
            pltpu.sync_copy(items_h.at[pl.ds(pbase, S)], idx_v)
            hs = []
            for k in range(KI):
                sl = pl.ds(k * IDX_CHUNK, IDX_CHUNK)
                idx = idx_v.at[sl]
                hs.append(pltpu.async_copy(iemb_h.at[idx], irows_v.at[sl], sem))
                hs.append(pltpu.async_copy(ibias_h.at[idx], ibias_v.at[sl], sem))
            for h in hs:
                h.wait()

            def group(g, c2):
                p_local = g * LANES + lid
                p_worker = sc * S + p_local
                u_loc = lax.div(p_worker, jnp.int32(L))
                acc = plsc.load_gather(ibias_v, [p_local])
                acc = acc + plsc.load_gather(ubias_v, [u_loc])
                for d in range(D):
                    dv = jnp.full((16,), d, jnp.int32)
                    ie = plsc.load_gather(irows_v, [p_local, dv])
                    ue = plsc.load_gather(uemb_v, [u_loc, dv])
                    acc = acc + ie * ue
                out_v[pl.ds(g * LANES, LANES)] = acc
                return c2

            lax.fori_loop(0, NG, group, 0, unroll=False)
            pltpu.sync_copy(out_v, out_h.at[pl.ds(pbase, S)])
            return carry

        lax.fori_loop(0, NSC, superchunk, 0, unroll=False)

    return mf_kernel


def kernel(users, items, user_emb, item_emb, user_bias, item_bias, bias):
    B, L = items.shape
    D = user_emb.shape[1]
    users = users.astype(jnp.int32)
    items_flat = items.astype(jnp.int32).reshape(-1)
    ubias2 = user_bias + bias[0]
    fn = _build_kernel(B, L, D, S=512)
    out_flat = fn(users, items_flat, user_emb, item_emb, ubias2, item_bias)
    return out_flat.reshape(B, L)


# speedup vs baseline: 1.2405x; 1.0721x over previous
import functools

import jax
import jax.numpy as jnp
from jax import lax
from jax.experimental import pallas as pl
from jax.experimental.pallas import tpu as pltpu
from jax.experimental.pallas import tpu_sc as plsc

NC = 2
NS = 16
LANES = 16
IDX_CHUNK = 128


def _build_kernel(B, L, D, S):
    NW = NC * NS
    UPW = B // NW
    PPW = UPW * L
    NSC = PPW // S
    KI = S // IDX_CHUNK
    KU = UPW // IDX_CHUNK
    NG = S // LANES
    assert NSC % 2 == 0 and NSC >= 4

    mesh = plsc.VectorSubcoreMesh(core_axis_name="c", subcore_axis_name="s")

    @functools.partial(
        pl.kernel,
        mesh=mesh,
        compiler_params=pltpu.CompilerParams(
            needs_layout_passes=False, use_tc_tiling_on_sc=False),
        out_type=jax.ShapeDtypeStruct((B * L,), jnp.float32),
        scratch_types=[
            pltpu.VMEM((UPW,), jnp.int32),
            pltpu.VMEM((UPW, D), jnp.float32),
            pltpu.VMEM((UPW,), jnp.float32),
            pltpu.VMEM((2, S), jnp.int32),
            pltpu.VMEM((2, S, D), jnp.float32),
            pltpu.VMEM((2, S), jnp.float32),
            pltpu.VMEM((2, S), jnp.float32),
            pltpu.SemaphoreType.DMA,
            pltpu.SemaphoreType.DMA,
            pltpu.SemaphoreType.DMA,
            pltpu.SemaphoreType.DMA,
            pltpu.SemaphoreType.DMA,
        ],
    )
    def mf_kernel(users_h, items_h, uemb_h, iemb_h, ubias_h, ibias_h, out_h,
                  uidx_v, uemb_v, ubias_v, idx_v, irows_v, ibias_v, out_v,
                  usem, gsem0, gsem1, osem0, osem1):
        w = lax.axis_index("s") * NC + lax.axis_index("c")
        ubase = w * UPW
        pbase0 = w * PPW
        gsem = (gsem0, gsem1)
        osem = (osem0, osem1)

        def fire_gathers(sc, b):
            pbase = pbase0 + sc * S
            pltpu.sync_copy(items_h.at[pl.ds(pbase, S)], idx_v.at[b])
            for k in range(KI):
                sl = pl.ds(k * IDX_CHUNK, IDX_CHUNK)
                idx = idx_v.at[b, sl]
                pltpu.async_copy(iemb_h.at[idx], irows_v.at[b, sl], gsem[b])
                pltpu.async_copy(ibias_h.at[idx], ibias_v.at[b, sl], gsem[b])

        def drain_gathers(sc, b):
            for k in range(KI):
                sl = pl.ds(k * IDX_CHUNK, IDX_CHUNK)
                idx = idx_v.at[b, sl]
                pltpu.make_async_copy(
                    iemb_h.at[idx], irows_v.at[b, sl], gsem[b]).wait()
                pltpu.make_async_copy(
                    ibias_h.at[idx], ibias_v.at[b, sl], gsem[b]).wait()

        pltpu.sync_copy(users_h.at[pl.ds(ubase, UPW)], uidx_v)
        for k in range(KU):
            sl = pl.ds(k * IDX_CHUNK, IDX_CHUNK)
            idx = uidx_v.at[sl]
            pltpu.async_copy(uemb_h.at[idx], uemb_v.at[sl], usem)
            pltpu.async_copy(ubias_h.at[idx], ubias_v.at[sl], usem)
        fire_gathers(jnp.int32(0), 0)
        fire_gathers(jnp.int32(1), 1)
        for k in range(KU):
            sl = pl.ds(k * IDX_CHUNK, IDX_CHUNK)
            idx = uidx_v.at[sl]
            pltpu.make_async_copy(uemb_h.at[idx], uemb_v.at[sl], usem).wait()
            pltpu.make_async_copy(ubias_h.at[idx], ubias_v.at[sl], usem).wait()

        lid = lax.iota(jnp.int32, 16)

        def body(sc, b):
            drain_gathers(sc, b)

            @pl.when(sc >= 2)
            def _():
                pltpu.make_async_copy(
                    out_v.at[b],
                    out_h.at[pl.ds(pbase0 + (sc - 2) * S, S)],
                    osem[b]).wait()

            def group(g, c2):
                p_local = g * LANES + lid
                p_worker = sc * S + p_local
                u_loc = lax.div(p_worker, jnp.int32(L))
                acc = plsc.load_gather(ibias_v.at[b], [p_local])
                acc = acc + plsc.load_gather(ubias_v, [u_loc])
                for d in range(D):
                    dv = jnp.full((16,), d, jnp.int32)
                    ie = plsc.load_gather(irows_v.at[b], [p_local, dv])
                    ue = plsc.load_gather(uemb_v, [u_loc, dv])
                    acc = acc + ie * ue
                out_v[b, pl.ds(g * LANES, LANES)] = acc
                return c2

            lax.fori_loop(0, NG, group, 0, unroll=False)
            pltpu.async_copy(
                out_v.at[b], out_h.at[pl.ds(pbase0 + sc * S, S)], osem[b])

            @pl.when(sc + 2 < NSC)
            def _():
                fire_gathers(sc + 2, b)

        def pair_body(sc2, carry):
            body(2 * sc2, 0)
            body(2 * sc2 + 1, 1)
            return carry

        lax.fori_loop(0, NSC // 2, pair_body, 0, unroll=False)

        for b in range(2):
            pltpu.make_async_copy(
                out_v.at[b],
                out_h.at[pl.ds(pbase0 + (NSC - 2 + b) * S, S)],
                osem[b]).wait()

    return mf_kernel


def kernel(users, items, user_emb, item_emb, user_bias, item_bias, bias):
    B, L = items.shape
    D = user_emb.shape[1]
    users = users.astype(jnp.int32)
    items_flat = items.astype(jnp.int32).reshape(-1)
    ubias2 = user_bias + bias[0]
    fn = _build_kernel(B, L, D, S=1280)
    out_flat = fn(users, items_flat, user_emb, item_emb, ubias2, item_bias)
    return out_flat.reshape(B, L)
